# baseline (speedup 1.0000x reference)
import functools

import jax
import jax.numpy as jnp
from jax import lax
from jax.experimental import pallas as pl
from jax.experimental.pallas import tpu as pltpu
from jax.experimental.pallas import tpu_sc as plsc

VOCAB = 1000000
D = 64
DP = 128
B = 4096
L = 50
N = B * L
NC, NS, LANES = 2, 16, 16
NW = NC * NS
PER_W = N // NW
CHUNK = 128
NCHUNK = PER_W // CHUNK

_EPS = 1e-8


def _normalize(rows_v, out_v):
    @plsc.parallel_loop(0, CHUNK, unroll=8)
    def _(r):
        vs = [rows_v[r, pl.ds(k * LANES, LANES)] for k in range(D // LANES)]
        sq = [v * v for v in vs]
        s = (sq[0] + sq[1]) + (sq[2] + sq[3])
        t = jnp.broadcast_to(jnp.sum(s), (LANES,))
        ibits = plsc.bitcast(t, jnp.int32)
        seed = jnp.int32(0x5F3759DF) - lax.shift_right_logical(ibits, 1)
        y = plsc.bitcast(seed, jnp.float32)
        for _ in range(2):
            y = y * (1.5 - 0.5 * t * y * y)
        m = 1.0 / (t * y + _EPS)
        for k in range(D // LANES):
            out_v[r, pl.ds(k * LANES, LANES)] = vs[k] * m


def _tec_body(x_hbm, table_hbm, out_hbm,
              idx_v, rows0, rows1, out0, out1, g0, g1, w0, w1):
    cid = lax.axis_index("c")
    sid = lax.axis_index("s")
    wid = sid * NC + cid
    base = wid * PER_W

    pltpu.sync_copy(x_hbm.at[wid], idx_v)

    pltpu.async_copy(table_hbm.at[idx_v.at[0]], rows0, g0)

    def step(c, rows_a, out_a, ga, wa, rows_b, out_b, gb, wb):
        @pl.when(c > 0)
        def _():
            pltpu.make_async_copy(out_b, out_hbm.at[pl.ds(0, CHUNK)], wb).wait()

        @pl.when(c + 1 < NCHUNK)
        def _():
            pltpu.async_copy(table_hbm.at[idx_v.at[c + 1]], rows_b, gb)

        pltpu.make_async_copy(table_hbm.at[idx_v.at[c]], rows_a, ga).wait()
        _normalize(rows_a, out_a)
        pltpu.async_copy(out_a, out_hbm.at[pl.ds(base + c * CHUNK, CHUNK)], wa)

    def pair_body(c2, carry):
        c = c2 * 2
        step(c, rows0, out0, g0, w0, rows1, out1, g1, w1)
        step(c + 1, rows1, out1, g1, w1, rows0, out0, g0, w0)
        return carry

    lax.fori_loop(0, NCHUNK // 2, pair_body, 0)

    pltpu.make_async_copy(out1, out_hbm.at[pl.ds(0, CHUNK)], w1).wait()


@jax.jit
def _embed_norm(x_grouped, table_padded):
    run = pl.kernel(
        _tec_body,
        out_type=jax.ShapeDtypeStruct((N, D), jnp.float32),
        mesh=plsc.VectorSubcoreMesh(
            core_axis_name="c", subcore_axis_name="s",
            num_cores=NC, num_subcores=NS,
        ),
        scratch_types=[
            pltpu.VMEM((NCHUNK, CHUNK), jnp.int32),
            pltpu.VMEM((CHUNK, D), jnp.float32),
            pltpu.VMEM((CHUNK, D), jnp.float32),
            pltpu.VMEM((CHUNK, D), jnp.float32),
            pltpu.VMEM((CHUNK, D), jnp.float32),
            pltpu.SemaphoreType.DMA,
            pltpu.SemaphoreType.DMA,
            pltpu.SemaphoreType.DMA,
            pltpu.SemaphoreType.DMA,
        ],
        compiler_params=pltpu.CompilerParams(
            needs_layout_passes=False,
            use_tc_tiling_on_sc=False,
        ),
    )
    return run(x_grouped, table_padded)


def kernel(x, lengths, embed_weight):
    table_padded = jnp.pad(embed_weight, ((0, 0), (0, DP - D)))
    table_view = table_padded.reshape(2 * VOCAB, D)
    x_grouped = x.reshape(NW, NCHUNK, CHUNK) * 2
    out = _embed_norm(x_grouped, table_view)
    emb = out.reshape(B, L, D)
    return (emb, lengths, emb)


# speedup vs baseline: 1.0058x; 1.0058x over previous
import functools

import jax
import jax.numpy as jnp
from jax import lax
from jax.experimental import pallas as pl
from jax.experimental.pallas import tpu as pltpu
from jax.experimental.pallas import tpu_sc as plsc

VOCAB = 1000000
D = 64
DP = 128
B = 4096
L = 50
N = B * L
NC, NS, LANES = 2, 16, 16
NW = NC * NS
PER_W = N // NW
CHUNK = 128
NCHUNK = PER_W // CHUNK

_EPS = 1e-8


def _normalize(rows_v, out_v):
    @plsc.parallel_loop(0, CHUNK, unroll=4)
    def _(r):
        vs = [rows_v[r, pl.ds(k * LANES, LANES)] for k in range(D // LANES)]
        sq = [v * v for v in vs]
        s = (sq[0] + sq[1]) + (sq[2] + sq[3])
        t = jnp.broadcast_to(jnp.sum(s), (LANES,))
        ibits = plsc.bitcast(t, jnp.int32)
        seed = jnp.int32(0x5F3759DF) - lax.shift_right_logical(ibits, 1)
        y = plsc.bitcast(seed, jnp.float32)
        for _ in range(2):
            y = y * (1.5 - 0.5 * t * y * y)
        m = 1.0 / (t * y + _EPS)
        for k in range(D // LANES):
            out_v[r, pl.ds(k * LANES, LANES)] = vs[k] * m


def _tec_body(x_hbm, table_hbm, out_hbm,
              idx_v, rows0, rows1, out0, out1, g0, g1, w0, w1):
    cid = lax.axis_index("c")
    sid = lax.axis_index("s")
    wid = sid * NC + cid
    base = wid * PER_W

    pltpu.sync_copy(x_hbm.at[wid], idx_v)

    pltpu.async_copy(table_hbm.at[idx_v.at[0]], rows0, g0)

    def step(c, rows_a, out_a, ga, wa, rows_b, out_b, gb, wb):
        @pl.when(c > 0)
        def _():
            pltpu.make_async_copy(out_b, out_hbm.at[pl.ds(0, CHUNK)], wb).wait()

        @pl.when(c + 1 < NCHUNK)
        def _():
            pltpu.async_copy(table_hbm.at[idx_v.at[c + 1]], rows_b, gb)

        pltpu.make_async_copy(table_hbm.at[idx_v.at[c]], rows_a, ga).wait()
        _normalize(rows_a, out_a)
        pltpu.async_copy(out_a, out_hbm.at[pl.ds(base + c * CHUNK, CHUNK)], wa)

    def pair_body(c2, carry):
        c = c2 * 2
        step(c, rows0, out0, g0, w0, rows1, out1, g1, w1)
        step(c + 1, rows1, out1, g1, w1, rows0, out0, g0, w0)
        return carry

    lax.fori_loop(0, NCHUNK // 2, pair_body, 0)

    pltpu.make_async_copy(out1, out_hbm.at[pl.ds(0, CHUNK)], w1).wait()


@jax.jit
def _embed_norm(x_grouped, table_padded):
    run = pl.kernel(
        _tec_body,
        out_type=jax.ShapeDtypeStruct((N, D), jnp.float32),
        mesh=plsc.VectorSubcoreMesh(
            core_axis_name="c", subcore_axis_name="s",
            num_cores=NC, num_subcores=NS,
        ),
        scratch_types=[
            pltpu.VMEM((NCHUNK, CHUNK), jnp.int32),
            pltpu.VMEM((CHUNK, D), jnp.float32),
            pltpu.VMEM((CHUNK, D), jnp.float32),
            pltpu.VMEM((CHUNK, D), jnp.float32),
            pltpu.VMEM((CHUNK, D), jnp.float32),
            pltpu.SemaphoreType.DMA,
            pltpu.SemaphoreType.DMA,
            pltpu.SemaphoreType.DMA,
            pltpu.SemaphoreType.DMA,
        ],
        compiler_params=pltpu.CompilerParams(
            needs_layout_passes=False,
            use_tc_tiling_on_sc=False,
        ),
    )
    return run(x_grouped, table_padded)


def kernel(x, lengths, embed_weight):
    table_padded = jnp.pad(embed_weight, ((0, 0), (0, DP - D)))
    table_view = table_padded.reshape(2 * VOCAB, D)
    x_grouped = x.reshape(NW, NCHUNK, CHUNK) * 2
    out = _embed_norm(x_grouped, table_view)
    emb = out.reshape(B, L, D)
    return (emb, lengths, emb)
